# baseline (speedup 1.0000x reference)
import functools

import jax
import jax.numpy as jnp
from jax import lax
from jax.experimental import pallas as pl
from jax.experimental.pallas import tpu as pltpu
from jax.experimental.pallas import tpu_sc as plsc

N = 10000
E = 320000
D = 128
HID = 256
OUTP = 128
H = 4
C = 32

NC = 2
NS = 16
L = 16
NW = NC * NS
EPT = E // NW
BLK = 80
NBLK = EPT // BLK
RQ = 624
RREM = N - NS * RQ


EBLK = E // 10


def _pre_body(x_ref, ln_g, ln_b, w1, b1, w2, b2, wres, bres, wgat,
              acats, acatd, ei_ref, hp_ref, ts_ref, td_ref, res_ref,
              src_ref, dst_ref, sem):
    i = pl.program_id(0)
    sl = pl.ds(i * EBLK, EBLK)
    cp1 = pltpu.make_async_copy(ei_ref.at[0, sl], src_ref.at[sl], sem)
    cp1.start()
    cp2 = pltpu.make_async_copy(ei_ref.at[1, sl], dst_ref.at[sl], sem)
    cp2.start()
    x = x_ref[...]
    mu = jnp.mean(x, axis=-1, keepdims=True)
    var = jnp.mean((x - mu) ** 2, axis=-1, keepdims=True)
    xn = (x - mu) * lax.rsqrt(var + 1e-5) * ln_g[...] + ln_b[...]
    hmid = jnp.maximum(jnp.dot(xn, w1[...], preferred_element_type=jnp.float32)
                       + b1[...], 0.0)
    h = jnp.dot(hmid, w2[...], preferred_element_type=jnp.float32) + b2[...]
    res_ref[...] = (jnp.dot(h, wres[...], preferred_element_type=jnp.float32)
                    + bres[...])
    hp = jnp.dot(h, wgat[...], preferred_element_type=jnp.float32)
    hp_ref[...] = hp
    ts_ref[...] = jnp.dot(hp, acats[...], preferred_element_type=jnp.float32)
    td_ref[...] = jnp.dot(hp, acatd[...], preferred_element_type=jnp.float32)
    cp1.wait()
    cp2.wait()


def _pre(x, ln_g, ln_b, w1, b1, w2, b2, wres, bres, wgat, acats, acatd, ei):
    blk = 1000
    grid = (N // blk,)
    full = lambda shape: pl.BlockSpec(shape, lambda i: (0,) * len(shape))
    return pl.pallas_call(
        _pre_body,
        grid=grid,
        in_specs=[
            pl.BlockSpec((blk, D), lambda i: (i, 0)),
            full((D,)), full((D,)),
            full((D, HID)), full((HID,)),
            full((HID, OUTP)), full((OUTP,)),
            full((OUTP, H * C)), full((H * C,)),
            full((OUTP, H * C)),
            full((H * C, L)), full((H * C, L)),
            pl.BlockSpec(memory_space=pl.ANY),
        ],
        out_specs=[
            pl.BlockSpec((blk, H * C), lambda i: (i, 0)),
            pl.BlockSpec((blk, L), lambda i: (i, 0)),
            pl.BlockSpec((blk, L), lambda i: (i, 0)),
            pl.BlockSpec((blk, H * C), lambda i: (i, 0)),
            pl.BlockSpec(memory_space=pl.ANY),
            pl.BlockSpec(memory_space=pl.ANY),
        ],
        out_shape=[
            jax.ShapeDtypeStruct((N, H * C), jnp.float32),
            jax.ShapeDtypeStruct((N, L), jnp.float32),
            jax.ShapeDtypeStruct((N, L), jnp.float32),
            jax.ShapeDtypeStruct((N, H * C), jnp.float32),
            jax.ShapeDtypeStruct((E,), jnp.int32),
            jax.ShapeDtypeStruct((E,), jnp.int32),
        ],
        scratch_shapes=[pltpu.SemaphoreType.DMA],
    )(x, ln_g, ln_b, w1, b1, w2, b2, wres, bres, wgat, acats, acatd, ei)


def _sc_edge_body(hp_hbm, ts_hbm, td_hbm, src_hbm, dst_hbm, z128_hbm, z16_hbm,
                  acc_out, den_out,
                  src_all, rows0, rows1, a0, a1, b0, b1, w0, w1,
                  dst0, dst1, dst2, dst3,
                  sg0, sg1, si0, si1, si2, si3, ss0, ss1, acc_sp, den_sp):
    c = lax.axis_index("c")
    s = lax.axis_index("s")
    rows_b = [rows0, rows1]
    a_b = [a0, a1]
    b_b = [b0, b1]
    w_b = [w0, w1]
    dst_b = [dst0, dst1, dst2, dst3]
    sg_b = [sg0, sg1]
    si_b = [si0, si1, si2, si3]
    ss_b = [ss0, ss1]
    pltpu.sync_copy(z128_hbm.at[pl.ds(0, RQ)], acc_sp.at[pl.ds(s * RQ, RQ)])
    pltpu.sync_copy(z16_hbm.at[pl.ds(0, RQ)], den_sp.at[pl.ds(s * RQ, RQ)])

    @pl.when(s == 0)
    def _():
        pltpu.sync_copy(z128_hbm.at[pl.ds(0, RREM)],
                        acc_sp.at[pl.ds(NS * RQ, RREM)])
        pltpu.sync_copy(z16_hbm.at[pl.ds(0, RREM)],
                        den_sp.at[pl.ds(NS * RQ, RREM)])

    wid = c * NS + s
    ebase = wid * EPT
    pltpu.sync_copy(src_hbm.at[pl.ds(ebase, EPT)], src_all)
    plsc.subcore_barrier()

    def prefetch_idx(i, q):
        pltpu.async_copy(dst_hbm.at[pl.ds(ebase + i * BLK, BLK)],
                         dst_b[q], si_b[q])

    def prefetch_gathers(i, p, q):
        pltpu.make_async_copy(dst_hbm.at[pl.ds(ebase + i * BLK, BLK)],
                              dst_b[q], si_b[q]).wait()
        sidx = src_all.at[pl.ds(i * BLK, BLK)]
        pltpu.async_copy(hp_hbm.at[sidx], rows_b[p], sg_b[p])
        pltpu.async_copy(ts_hbm.at[sidx], a_b[p], sg_b[p])
        pltpu.async_copy(td_hbm.at[dst_b[q]], b_b[p], sg_b[p])

    def process(i, p, q):
        sidx = src_all.at[pl.ds(i * BLK, BLK)]
        pltpu.make_async_copy(hp_hbm.at[sidx], rows_b[p], sg_b[p]).wait()
        pltpu.make_async_copy(ts_hbm.at[sidx], a_b[p], sg_b[p]).wait()
        pltpu.make_async_copy(td_hbm.at[dst_b[q]], b_b[p], sg_b[p]).wait()
        rows_v, a_v, b_v, w_v, dst_v = (rows_b[p], a_b[p], b_b[p], w_b[p],
                                        dst_b[q])

        @plsc.parallel_loop(0, BLK, unroll=8)
        def _(e):
            lg = a_v[e, :] + b_v[e, :]
            lg = jnp.where(lg > 0, lg, 0.2 * lg)
            w_v[e, :] = jnp.exp(lg)

        @plsc.parallel_loop(0, BLK, unroll=4)
        def _(e):
            wrow = w_v[e, :]
            for k in range(H * C // L):
                wk = wrow[k // (C // L)]
                rows_v[e, pl.ds(k * L, L)] = rows_v[e, pl.ds(k * L, L)] * wk

        d1 = pltpu.async_copy(rows_v, acc_sp.at[dst_v], ss_b[p], add=True)
        pltpu.sync_copy(w_v, den_sp.at[dst_v], add=True)
        d1.wait()

    prefetch_idx(0, 0)
    prefetch_idx(1, 1)
    prefetch_gathers(0, 0, 0)

    def quad(j, carry):
        for qq in range(4):
            i4 = 4 * j + qq

            @pl.when(i4 + 2 < NBLK)
            def _():
                prefetch_idx(i4 + 2, (qq + 2) % 4)

            prefetch_gathers(i4 + 1, (qq + 1) % 2, (qq + 1) % 4)
            process(i4, qq % 2, qq % 4)
        return carry

    lax.fori_loop(0, (NBLK - 1) // 4, quad, 0, unroll=False)
    process(NBLK - 1, (NBLK - 1) % 2, (NBLK - 1) % 4)
    plsc.subcore_barrier()
    sl = pl.ds(s * RQ, RQ)
    pltpu.sync_copy(acc_sp.at[sl], acc_out.at[c, sl])
    pltpu.sync_copy(den_sp.at[sl], den_out.at[c, sl])

    @pl.when(s == 0)
    def _():
        sl2 = pl.ds(NS * RQ, RREM)
        pltpu.sync_copy(acc_sp.at[sl2], acc_out.at[c, sl2])
        pltpu.sync_copy(den_sp.at[sl2], den_out.at[c, sl2])


def _sc_edge(hp, ts, td, src, dst, z128, z16):
    mesh = plsc.VectorSubcoreMesh(core_axis_name="c", subcore_axis_name="s")
    kern = functools.partial(
        pl.kernel,
        mesh=mesh,
        compiler_params=pltpu.CompilerParams(use_tc_tiling_on_sc=False),
        out_type=[
            jax.ShapeDtypeStruct((NC, N, H * C), jnp.float32),
            jax.ShapeDtypeStruct((NC, N, L), jnp.float32),
        ],
        scratch_types=(
            [pltpu.VMEM((EPT,), jnp.int32)]
            + [pltpu.VMEM((BLK, H * C), jnp.float32)] * 2
            + [pltpu.VMEM((BLK, L), jnp.float32)] * 6
            + [pltpu.VMEM((BLK,), jnp.int32)] * 4
            + [pltpu.SemaphoreType.DMA] * 8
            + [
                pltpu.VMEM_SHARED((N, H * C), jnp.float32),
                pltpu.VMEM_SHARED((N, L), jnp.float32),
            ]
        ),
    )(_sc_edge_body)
    return kern(hp, ts, td, src, dst, z128, z16)


def _post_body(acc0, acc1, den0, den1, ts, td, hp, res, gbias, pexp, out_ref):
    lg = ts[...][:, :H] + td[...][:, :H]
    lg = jnp.where(lg > 0, lg, 0.2 * lg)
    wself = jnp.exp(lg)
    wexp = jnp.dot(wself, pexp[...], preferred_element_type=jnp.float32)
    num = acc0[...] + acc1[...] + hp[...] * wexp
    dself = jnp.dot((den0[...] + den1[...])[:, :H] + wself, pexp[...],
                    preferred_element_type=jnp.float32)
    g = num / (dself + 1e-16) + gbias[...]
    out_ref[...] = (jnp.where(g > 0, g, jnp.exp(jnp.minimum(g, 0.0)) - 1.0)
                    + res[...])


def _post(acc, den, ts, td, hp, res, gbias, pexp):
    blk = 1000
    grid = (N // blk,)
    full = lambda shape: pl.BlockSpec(shape, lambda i: (0,) * len(shape))
    return pl.pallas_call(
        _post_body,
        grid=grid,
        in_specs=[
            pl.BlockSpec((blk, H * C), lambda i: (i, 0)),
            pl.BlockSpec((blk, H * C), lambda i: (i, 0)),
            pl.BlockSpec((blk, L), lambda i: (i, 0)),
            pl.BlockSpec((blk, L), lambda i: (i, 0)),
            pl.BlockSpec((blk, L), lambda i: (i, 0)),
            pl.BlockSpec((blk, L), lambda i: (i, 0)),
            pl.BlockSpec((blk, H * C), lambda i: (i, 0)),
            pl.BlockSpec((blk, H * C), lambda i: (i, 0)),
            full((H * C,)),
            full((H, H * C)),
        ],
        out_specs=pl.BlockSpec((blk, H * C), lambda i: (i, 0)),
        out_shape=jax.ShapeDtypeStruct((N, H * C), jnp.float32),
    )(acc[0], acc[1], den[0], den[1], ts, td, hp, res, gbias, pexp)


def kernel(x, edge_index, ln_g, ln_b, W1, b1, W2, b2, Wres, bres, Wgat,
           att_src, att_dst, gat_bias):
    f32 = jnp.float32
    eye = jnp.repeat(jnp.eye(H, L, dtype=f32), C, axis=0)
    acats = eye * att_src.reshape(H * C, 1)
    acatd = eye * att_dst.reshape(H * C, 1)
    pexp = jnp.repeat(jnp.eye(H, dtype=f32), C, axis=1)

    hp, ts, td, res, src, dst = _pre(x, ln_g, ln_b, W1, b1, W2, b2, Wres,
                                     bres, Wgat, acats, acatd, edge_index)

    z128 = jnp.zeros((RQ, H * C), f32)
    z16 = jnp.zeros((RQ, L), f32)
    acc, den = _sc_edge(hp, ts, td, src, dst, z128, z16)

    return _post(acc, den, ts, td, hp, res, gat_bias, pexp)


# speedup vs baseline: 1.2532x; 1.2532x over previous
import functools

import jax
import jax.numpy as jnp
from jax import lax
from jax.experimental import pallas as pl
from jax.experimental.pallas import tpu as pltpu
from jax.experimental.pallas import tpu_sc as plsc

N = 10000
E = 320000
D = 128
HID = 256
OUTP = 128
H = 4
C = 32

NC = 2
NS = 16
L = 16
NW = NC * NS
EPT = E // NW
BLK = 80
NBLK = EPT // BLK
RQ = 624
RREM = N - NS * RQ


def _pre_body(x_ref, ln_g, ln_b, w1, b1, w2, b2, wres, bres, wgat,
              acats, acatd, hp_ref, ts_ref, td_ref, res_ref):
    x = x_ref[...]
    mu = jnp.mean(x, axis=-1, keepdims=True)
    var = jnp.mean((x - mu) ** 2, axis=-1, keepdims=True)
    xn = (x - mu) * lax.rsqrt(var + 1e-5) * ln_g[...] + ln_b[...]
    hmid = jnp.maximum(jnp.dot(xn, w1[...], preferred_element_type=jnp.float32)
                       + b1[...], 0.0)
    h = jnp.dot(hmid, w2[...], preferred_element_type=jnp.float32) + b2[...]
    res_ref[...] = (jnp.dot(h, wres[...], preferred_element_type=jnp.float32)
                    + bres[...])
    hp = jnp.dot(h, wgat[...], preferred_element_type=jnp.float32)
    hp_ref[...] = hp
    ts_ref[...] = jnp.dot(hp, acats[...], preferred_element_type=jnp.float32)
    td_ref[...] = jnp.dot(hp, acatd[...], preferred_element_type=jnp.float32)


def _pre(x, ln_g, ln_b, w1, b1, w2, b2, wres, bres, wgat, acats, acatd):
    blk = 1000
    grid = (N // blk,)
    full = lambda shape: pl.BlockSpec(shape, lambda i: (0,) * len(shape))
    return pl.pallas_call(
        _pre_body,
        grid=grid,
        in_specs=[
            pl.BlockSpec((blk, D), lambda i: (i, 0)),
            full((D,)), full((D,)),
            full((D, HID)), full((HID,)),
            full((HID, OUTP)), full((OUTP,)),
            full((OUTP, H * C)), full((H * C,)),
            full((OUTP, H * C)),
            full((H * C, L)), full((H * C, L)),
        ],
        out_specs=[
            pl.BlockSpec((blk, H * C), lambda i: (i, 0)),
            pl.BlockSpec((blk, L), lambda i: (i, 0)),
            pl.BlockSpec((blk, L), lambda i: (i, 0)),
            pl.BlockSpec((blk, H * C), lambda i: (i, 0)),
        ],
        out_shape=[
            jax.ShapeDtypeStruct((N, H * C), jnp.float32),
            jax.ShapeDtypeStruct((N, L), jnp.float32),
            jax.ShapeDtypeStruct((N, L), jnp.float32),
            jax.ShapeDtypeStruct((N, H * C), jnp.float32),
        ],
    )(x, ln_g, ln_b, w1, b1, w2, b2, wres, bres, wgat, acats, acatd)


def _sc_edge_body(hp_hbm, ts_hbm, td_hbm, src_hbm, dst_hbm, z128_hbm, z16_hbm,
                  acc_out, den_out,
                  src_all, rows0, rows1, a0, a1, b0, b1, w0, w1,
                  dst0, dst1, dst2, dst3,
                  sg0, sg1, si0, si1, si2, si3, ss0, ss1, acc_sp, den_sp):
    c = lax.axis_index("c")
    s = lax.axis_index("s")
    rows_b = [rows0, rows1]
    a_b = [a0, a1]
    b_b = [b0, b1]
    w_b = [w0, w1]
    dst_b = [dst0, dst1, dst2, dst3]
    sg_b = [sg0, sg1]
    si_b = [si0, si1, si2, si3]
    ss_b = [ss0, ss1]
    pltpu.sync_copy(z128_hbm.at[pl.ds(0, RQ)], acc_sp.at[pl.ds(s * RQ, RQ)])
    pltpu.sync_copy(z16_hbm.at[pl.ds(0, RQ)], den_sp.at[pl.ds(s * RQ, RQ)])

    @pl.when(s == 0)
    def _():
        pltpu.sync_copy(z128_hbm.at[pl.ds(0, RREM)],
                        acc_sp.at[pl.ds(NS * RQ, RREM)])
        pltpu.sync_copy(z16_hbm.at[pl.ds(0, RREM)],
                        den_sp.at[pl.ds(NS * RQ, RREM)])

    wid = c * NS + s
    ebase = wid * EPT
    pltpu.sync_copy(src_hbm.at[pl.ds(ebase, EPT)], src_all)
    plsc.subcore_barrier()

    def prefetch_idx(i, q):
        pltpu.async_copy(dst_hbm.at[pl.ds(ebase + i * BLK, BLK)],
                         dst_b[q], si_b[q])

    def prefetch_gathers(i, p, q):
        pltpu.make_async_copy(dst_hbm.at[pl.ds(ebase + i * BLK, BLK)],
                              dst_b[q], si_b[q]).wait()
        sidx = src_all.at[pl.ds(i * BLK, BLK)]
        pltpu.async_copy(hp_hbm.at[sidx], rows_b[p], sg_b[p])
        pltpu.async_copy(ts_hbm.at[sidx], a_b[p], sg_b[p])
        pltpu.async_copy(td_hbm.at[dst_b[q]], b_b[p], sg_b[p])

    def process(i, p, q):
        sidx = src_all.at[pl.ds(i * BLK, BLK)]
        pltpu.make_async_copy(hp_hbm.at[sidx], rows_b[p], sg_b[p]).wait()
        pltpu.make_async_copy(ts_hbm.at[sidx], a_b[p], sg_b[p]).wait()
        pltpu.make_async_copy(td_hbm.at[dst_b[q]], b_b[p], sg_b[p]).wait()
        rows_v, a_v, b_v, w_v, dst_v = (rows_b[p], a_b[p], b_b[p], w_b[p],
                                        dst_b[q])

        def fused(lo, hi):
            @plsc.parallel_loop(lo, hi, unroll=4)
            def _(e):
                lg = a_v[e, :] + b_v[e, :]
                lg = jnp.where(lg > 0, lg, 0.2 * lg)
                wv = jnp.exp(lg)
                w_v[e, :] = wv
                for k in range(H * C // L):
                    wk = wv[k // (C // L)]
                    rows_v[e, pl.ds(k * L, L)] = rows_v[e, pl.ds(k * L, L)] * wk

        HB = BLK // 2
        fused(0, HB)
        d1 = pltpu.async_copy(rows_v.at[pl.ds(0, HB)],
                              acc_sp.at[dst_v.at[pl.ds(0, HB)]],
                              ss_b[p], add=True)
        fused(HB, BLK)
        d2 = pltpu.async_copy(rows_v.at[pl.ds(HB, HB)],
                              acc_sp.at[dst_v.at[pl.ds(HB, HB)]],
                              ss_b[p], add=True)
        pltpu.sync_copy(w_v, den_sp.at[dst_v], add=True)
        d1.wait()
        d2.wait()

    prefetch_idx(0, 0)
    prefetch_idx(1, 1)
    prefetch_gathers(0, 0, 0)

    def quad(j, carry):
        for qq in range(4):
            i4 = 4 * j + qq

            @pl.when(i4 + 2 < NBLK)
            def _():
                prefetch_idx(i4 + 2, (qq + 2) % 4)

            prefetch_gathers(i4 + 1, (qq + 1) % 2, (qq + 1) % 4)
            process(i4, qq % 2, qq % 4)
        return carry

    lax.fori_loop(0, (NBLK - 1) // 4, quad, 0, unroll=False)
    process(NBLK - 1, (NBLK - 1) % 2, (NBLK - 1) % 4)
    plsc.subcore_barrier()
    sl = pl.ds(s * RQ, RQ)
    pltpu.sync_copy(acc_sp.at[sl], acc_out.at[c, sl])
    pltpu.sync_copy(den_sp.at[sl], den_out.at[c, sl])

    @pl.when(s == 0)
    def _():
        sl2 = pl.ds(NS * RQ, RREM)
        pltpu.sync_copy(acc_sp.at[sl2], acc_out.at[c, sl2])
        pltpu.sync_copy(den_sp.at[sl2], den_out.at[c, sl2])


def _sc_edge(hp, ts, td, src, dst, z128, z16):
    mesh = plsc.VectorSubcoreMesh(core_axis_name="c", subcore_axis_name="s")
    kern = functools.partial(
        pl.kernel,
        mesh=mesh,
        compiler_params=pltpu.CompilerParams(use_tc_tiling_on_sc=False),
        out_type=[
            jax.ShapeDtypeStruct((NC, N, H * C), jnp.float32),
            jax.ShapeDtypeStruct((NC, N, L), jnp.float32),
        ],
        scratch_types=(
            [pltpu.VMEM((EPT,), jnp.int32)]
            + [pltpu.VMEM((BLK, H * C), jnp.float32)] * 2
            + [pltpu.VMEM((BLK, L), jnp.float32)] * 6
            + [pltpu.VMEM((BLK,), jnp.int32)] * 4
            + [pltpu.SemaphoreType.DMA] * 8
            + [
                pltpu.VMEM_SHARED((N, H * C), jnp.float32),
                pltpu.VMEM_SHARED((N, L), jnp.float32),
            ]
        ),
    )(_sc_edge_body)
    return kern(hp, ts, td, src, dst, z128, z16)


def _post_body(acc0, acc1, den0, den1, ts, td, hp, res, gbias, pexp, out_ref):
    lg = ts[...][:, :H] + td[...][:, :H]
    lg = jnp.where(lg > 0, lg, 0.2 * lg)
    wself = jnp.exp(lg)
    wexp = jnp.dot(wself, pexp[...], preferred_element_type=jnp.float32)
    num = acc0[...] + acc1[...] + hp[...] * wexp
    dself = jnp.dot((den0[...] + den1[...])[:, :H] + wself, pexp[...],
                    preferred_element_type=jnp.float32)
    g = num / (dself + 1e-16) + gbias[...]
    out_ref[...] = (jnp.where(g > 0, g, jnp.exp(jnp.minimum(g, 0.0)) - 1.0)
                    + res[...])


def _post(acc, den, ts, td, hp, res, gbias, pexp):
    blk = 1000
    grid = (N // blk,)
    full = lambda shape: pl.BlockSpec(shape, lambda i: (0,) * len(shape))
    return pl.pallas_call(
        _post_body,
        grid=grid,
        in_specs=[
            pl.BlockSpec((blk, H * C), lambda i: (i, 0)),
            pl.BlockSpec((blk, H * C), lambda i: (i, 0)),
            pl.BlockSpec((blk, L), lambda i: (i, 0)),
            pl.BlockSpec((blk, L), lambda i: (i, 0)),
            pl.BlockSpec((blk, L), lambda i: (i, 0)),
            pl.BlockSpec((blk, L), lambda i: (i, 0)),
            pl.BlockSpec((blk, H * C), lambda i: (i, 0)),
            pl.BlockSpec((blk, H * C), lambda i: (i, 0)),
            full((H * C,)),
            full((H, H * C)),
        ],
        out_specs=pl.BlockSpec((blk, H * C), lambda i: (i, 0)),
        out_shape=jax.ShapeDtypeStruct((N, H * C), jnp.float32),
    )(acc[0], acc[1], den[0], den[1], ts, td, hp, res, gbias, pexp)


def kernel(x, edge_index, ln_g, ln_b, W1, b1, W2, b2, Wres, bres, Wgat,
           att_src, att_dst, gat_bias):
    f32 = jnp.float32
    eye = jnp.repeat(jnp.eye(H, L, dtype=f32), C, axis=0)
    acats = eye * att_src.reshape(H * C, 1)
    acatd = eye * att_dst.reshape(H * C, 1)
    pexp = jnp.repeat(jnp.eye(H, dtype=f32), C, axis=1)

    hp, ts, td, res = _pre(x, ln_g, ln_b, W1, b1, W2, b2, Wres, bres, Wgat,
                           acats, acatd)

    src = edge_index[0]
    dst = edge_index[1]
    z128 = jnp.zeros((RQ, H * C), f32)
    z16 = jnp.zeros((RQ, L), f32)
    acc, den = _sc_edge(hp, ts, td, src, dst, z128, z16)

    return _post(acc, den, ts, td, hp, res, gat_bias, pexp)
